# Initial kernel scaffold; baseline (speedup 1.0000x reference)
import jax
import jax.numpy as jnp
from jax.experimental import pallas as pl


def kernel(x, edge_index, cj, ci, W, b):
    raise NotImplementedError("write your pallas kernel here")


# speedup vs baseline: 6.4849x; 6.4849x over previous
import functools

import jax
import jax.numpy as jnp
from jax import lax
from jax.experimental import pallas as pl
from jax.experimental.pallas import tpu as pltpu
from jax.experimental.pallas import tpu_sc as plsc

N_NODES = 10000
D = 128
E = 320000

CHUNK = 128
N_CHUNKS = E // CHUNK
NW = 32
MAIN_ROWS = 624
TAIL_ROWS = 8
BASE_CH = N_CHUNKS // NW
EXTRA = N_CHUNKS - BASE_CH * NW

M_TILE = 2000


def _mm_body(x_ref, w_ref, b_ref, cj_ref, o_ref):
    h = lax.dot_general(x_ref[...], w_ref[...],
                        (((1,), (1,)), ((), ())),
                        preferred_element_type=jnp.float32)
    o_ref[...] = (h + b_ref[...]) * cj_ref[...]


def _feature_transform(x, w, b, cj):
    grid = (N_NODES // M_TILE,)
    return pl.pallas_call(
        _mm_body,
        grid=grid,
        in_specs=[
            pl.BlockSpec((M_TILE, D), lambda i: (i, 0)),
            pl.BlockSpec((D, D), lambda i: (0, 0)),
            pl.BlockSpec((1, D), lambda i: (0, 0)),
            pl.BlockSpec((M_TILE, 1), lambda i: (i, 0)),
        ],
        out_specs=pl.BlockSpec((M_TILE, D), lambda i: (i, 0)),
        out_shape=jax.ShapeDtypeStruct((N_NODES, D), jnp.float32),
    )(x, w, b, cj)


def _sc_scatter(weighted, src, dst, zeros):
    mesh = plsc.VectorSubcoreMesh(core_axis_name="c", subcore_axis_name="s")

    @functools.partial(
        pl.kernel,
        mesh=mesh,
        out_type=jax.ShapeDtypeStruct((2, N_NODES, D), jnp.float32),
        scratch_types=[
            pltpu.VMEM((CHUNK,), jnp.int32),
            pltpu.VMEM((CHUNK,), jnp.int32),
            pltpu.VMEM((CHUNK, D), jnp.float32),
            pltpu.VMEM_SHARED((N_NODES, D), jnp.float32),
            pltpu.SemaphoreType.DMA,
        ],
    )
    def k(w_hbm, src_hbm, dst_hbm, z_hbm, out_hbm, src_v, dst_v, rows_v, acc_sh, sem):
        cid = lax.axis_index("c")
        sid = lax.axis_index("s")
        wid = sid * 2 + cid

        pltpu.sync_copy(z_hbm.at[pl.ds(sid * MAIN_ROWS, MAIN_ROWS)],
                        acc_sh.at[pl.ds(sid * MAIN_ROWS, MAIN_ROWS)])

        @pl.when(sid < 2)
        def _():
            off = 16 * MAIN_ROWS + sid * TAIL_ROWS
            pltpu.sync_copy(z_hbm.at[pl.ds(off, TAIL_ROWS)],
                            acc_sh.at[pl.ds(off, TAIL_ROWS)])

        plsc.subcore_barrier()

        base = BASE_CH * wid + jnp.minimum(wid, EXTRA)
        n = BASE_CH + jnp.where(wid < EXTRA, 1, 0)

        def body(i, carry):
            g = base + i
            pltpu.sync_copy(src_hbm.at[pl.ds(g * CHUNK, CHUNK)], src_v)
            pltpu.sync_copy(dst_hbm.at[pl.ds(g * CHUNK, CHUNK)], dst_v)
            pltpu.async_copy(w_hbm.at[src_v], rows_v, sem).wait()
            pltpu.sync_copy(rows_v, acc_sh.at[dst_v], add=True)
            return carry

        lax.fori_loop(0, n, body, 0)

        plsc.subcore_barrier()
        pltpu.sync_copy(acc_sh.at[pl.ds(sid * MAIN_ROWS, MAIN_ROWS)],
                        out_hbm.at[cid, pl.ds(sid * MAIN_ROWS, MAIN_ROWS)])

        @pl.when(sid < 2)
        def _():
            off = 16 * MAIN_ROWS + sid * TAIL_ROWS
            pltpu.sync_copy(acc_sh.at[pl.ds(off, TAIL_ROWS)],
                            out_hbm.at[cid, pl.ds(off, TAIL_ROWS)])

    return k(weighted, src, dst, zeros)


def _combine_body(p_ref, ci_ref, o_ref):
    o_ref[...] = (p_ref[0] + p_ref[1]) * ci_ref[...]


def _combine(partials, ci):
    grid = (N_NODES // M_TILE,)
    return pl.pallas_call(
        _combine_body,
        grid=grid,
        in_specs=[
            pl.BlockSpec((2, M_TILE, D), lambda i: (0, i, 0)),
            pl.BlockSpec((M_TILE, 1), lambda i: (i, 0)),
        ],
        out_specs=pl.BlockSpec((M_TILE, D), lambda i: (i, 0)),
        out_shape=jax.ShapeDtypeStruct((N_NODES, D), jnp.float32),
    )(partials, ci)


def kernel(x, edge_index, cj, ci, W, b):
    src = edge_index[0].astype(jnp.int32)
    dst = edge_index[1].astype(jnp.int32)
    weighted = _feature_transform(x, W, b.reshape(1, D), cj)
    partials = _sc_scatter(weighted, src, dst,
                           jnp.zeros((N_NODES, D), jnp.float32))
    return _combine(partials, ci)


# speedup vs baseline: 10.2968x; 1.5878x over previous
import functools

import jax
import jax.numpy as jnp
from jax import lax
from jax.experimental import pallas as pl
from jax.experimental.pallas import tpu as pltpu
from jax.experimental.pallas import tpu_sc as plsc

N_NODES = 10000
D = 128
E = 320000

CHUNK = 128
N_CHUNKS = E // CHUNK
NW = 32
MAIN_ROWS = 624
TAIL_ROWS = 8
PAD_E = 512
N_CHUNKS_PAD = (E + PAD_E) // CHUNK
CH_A = 80
CH_B = 72
KI = 40
N_ACC = N_NODES + 8

M_TILE = 2000


def _mm_body(x_ref, w_ref, b_ref, cj_ref, o_ref):
    h = lax.dot_general(x_ref[...], w_ref[...],
                        (((1,), (1,)), ((), ())),
                        preferred_element_type=jnp.float32)
    o_ref[...] = (h + b_ref[...]) * cj_ref[...]


def _feature_transform(x, w, b, cj):
    grid = (N_NODES // M_TILE,)
    return pl.pallas_call(
        _mm_body,
        grid=grid,
        in_specs=[
            pl.BlockSpec((M_TILE, D), lambda i: (i, 0)),
            pl.BlockSpec((D, D), lambda i: (0, 0)),
            pl.BlockSpec((1, D), lambda i: (0, 0)),
            pl.BlockSpec((M_TILE, 1), lambda i: (i, 0)),
        ],
        out_specs=pl.BlockSpec((M_TILE, D), lambda i: (i, 0)),
        out_shape=jax.ShapeDtypeStruct((N_NODES, D), jnp.float32),
    )(x, w, b, cj)


def _sc_scatter(weighted, src2d, dst2d, zeros):
    mesh = plsc.VectorSubcoreMesh(core_axis_name="c", subcore_axis_name="s")

    @functools.partial(
        pl.kernel,
        mesh=mesh,
        out_type=jax.ShapeDtypeStruct((2, N_NODES, D), jnp.float32),
        scratch_types=[
            pltpu.VMEM((KI, CHUNK), jnp.int32),
            pltpu.VMEM((KI, CHUNK), jnp.int32),
            pltpu.VMEM((CHUNK,), jnp.int32),
            pltpu.VMEM((CHUNK,), jnp.int32),
            pltpu.VMEM((CHUNK, D), jnp.float32),
            pltpu.VMEM((CHUNK, D), jnp.float32),
            pltpu.VMEM_SHARED((N_ACC, D), jnp.float32),
            pltpu.SemaphoreType.DMA,
            pltpu.SemaphoreType.DMA,
        ],
    )
    def k(w_hbm, src_hbm, dst_hbm, z_hbm, out_hbm,
          src_v, dst_v, dstf0, dstf1, rows0, rows1, acc_sh, sem0, sem1):
        cid = lax.axis_index("c")
        sid = lax.axis_index("s")
        wid = sid * 2 + cid

        pltpu.sync_copy(z_hbm.at[pl.ds(sid * MAIN_ROWS, MAIN_ROWS)],
                        acc_sh.at[pl.ds(sid * MAIN_ROWS, MAIN_ROWS)])

        @pl.when(sid < 2)
        def _():
            off = 16 * MAIN_ROWS + sid * TAIL_ROWS
            pltpu.sync_copy(z_hbm.at[pl.ds(off, TAIL_ROWS)],
                            acc_sh.at[pl.ds(off, TAIL_ROWS)])

        base = pl.multiple_of(
            jnp.where(wid < 25, CH_A * wid, 25 * CH_A + CH_B * (wid - 25)), 8)
        n = jnp.where(wid < 25, CH_A, CH_B)

        plsc.subcore_barrier()

        for b in range(2):
            pltpu.sync_copy(src_hbm.at[pl.ds(base + b * KI, KI)], src_v)
            pltpu.sync_copy(dst_hbm.at[pl.ds(base + b * KI, KI)], dst_v)
            nb = KI if b == 0 else n - KI

            def stage_row(row, dstref):
                for t in range(CHUNK // 16):
                    dstref[pl.ds(16 * t, 16)] = dst_v[row, pl.ds(16 * t, 16)]

            pltpu.async_copy(w_hbm.at[src_v.at[0]], rows0, sem0)
            stage_row(0, dstf0)

            def body(h, carry):
                j = 2 * h
                pltpu.async_copy(w_hbm.at[src_v.at[j + 1]], rows1, sem1)
                stage_row(j + 1, dstf1)
                pltpu.make_async_copy(w_hbm.at[src_v.at[j]], rows0, sem0).wait()
                pltpu.sync_copy(rows0, acc_sh.at[dstf0], add=True)
                jn = jnp.minimum(j + 2, nb - 1)
                pltpu.async_copy(w_hbm.at[src_v.at[jn]], rows0, sem0)
                stage_row(jn, dstf0)
                pltpu.make_async_copy(w_hbm.at[src_v.at[j + 1]], rows1, sem1).wait()
                pltpu.sync_copy(rows1, acc_sh.at[dstf1], add=True)
                return carry

            lax.fori_loop(0, nb // 2, body, 0)
            pltpu.make_async_copy(w_hbm.at[src_v.at[0]], rows0, sem0).wait()

        plsc.subcore_barrier()
        pltpu.sync_copy(acc_sh.at[pl.ds(sid * MAIN_ROWS, MAIN_ROWS)],
                        out_hbm.at[cid, pl.ds(sid * MAIN_ROWS, MAIN_ROWS)])

        @pl.when(sid < 2)
        def _():
            off = 16 * MAIN_ROWS + sid * TAIL_ROWS
            pltpu.sync_copy(acc_sh.at[pl.ds(off, TAIL_ROWS)],
                            out_hbm.at[cid, pl.ds(off, TAIL_ROWS)])

    return k(weighted, src2d, dst2d, zeros)


def _combine_body(p_ref, ci_ref, o_ref):
    o_ref[...] = (p_ref[0] + p_ref[1]) * ci_ref[...]


def _combine(partials, ci):
    grid = (N_NODES // M_TILE,)
    return pl.pallas_call(
        _combine_body,
        grid=grid,
        in_specs=[
            pl.BlockSpec((2, M_TILE, D), lambda i: (0, i, 0)),
            pl.BlockSpec((M_TILE, 1), lambda i: (i, 0)),
        ],
        out_specs=pl.BlockSpec((M_TILE, D), lambda i: (i, 0)),
        out_shape=jax.ShapeDtypeStruct((N_NODES, D), jnp.float32),
    )(partials, ci)


def kernel(x, edge_index, cj, ci, W, b):
    src = jnp.concatenate(
        [edge_index[0].astype(jnp.int32),
         jnp.zeros((PAD_E,), jnp.int32)]).reshape(N_CHUNKS_PAD, CHUNK)
    dst = jnp.concatenate(
        [edge_index[1].astype(jnp.int32),
         N_NODES + (jnp.arange(PAD_E, dtype=jnp.int32) % 8)]
    ).reshape(N_CHUNKS_PAD, CHUNK)
    weighted = _feature_transform(x, W, b.reshape(1, D), cj)
    partials = _sc_scatter(weighted, src, dst,
                           jnp.zeros((N_NODES, D), jnp.float32))
    return _combine(partials, ci)


# speedup vs baseline: 10.6002x; 1.0295x over previous
import functools

import jax
import jax.numpy as jnp
from jax import lax
from jax.experimental import pallas as pl
from jax.experimental.pallas import tpu as pltpu
from jax.experimental.pallas import tpu_sc as plsc

N_NODES = 10000
D = 128
E = 320000

CHUNK = 128
N_CHUNKS = E // CHUNK
NW = 32
MAIN_ROWS = 624
TAIL_ROWS = 8
PAD_E = 512
N_CHUNKS_PAD = (E + PAD_E) // CHUNK
CH_A = 80
CH_B = 72
KI = 40
N_ACC = N_NODES + 8

M_TILE = 2000


def _mm_body(x_ref, w_ref, b_ref, cj_ref, o_ref):
    h = lax.dot_general(x_ref[...], w_ref[...],
                        (((1,), (1,)), ((), ())),
                        preferred_element_type=jnp.float32)
    o_ref[...] = (h + b_ref[...]) * cj_ref[...]


def _feature_transform(x, w, b, cj):
    grid = (N_NODES // M_TILE,)
    return pl.pallas_call(
        _mm_body,
        grid=grid,
        in_specs=[
            pl.BlockSpec((M_TILE, D), lambda i: (i, 0)),
            pl.BlockSpec((D, D), lambda i: (0, 0)),
            pl.BlockSpec((1, D), lambda i: (0, 0)),
            pl.BlockSpec((M_TILE, 1), lambda i: (i, 0)),
        ],
        out_specs=pl.BlockSpec((M_TILE, D), lambda i: (i, 0)),
        out_shape=jax.ShapeDtypeStruct((N_NODES, D), jnp.float32),
    )(x, w, b, cj)


def _sc_scatter(weighted, src2d, dst2d):
    mesh = plsc.VectorSubcoreMesh(core_axis_name="c", subcore_axis_name="s")

    @functools.partial(
        pl.kernel,
        mesh=mesh,
        out_type=jax.ShapeDtypeStruct((2, N_NODES, D), jnp.float32),
        scratch_types=[
            pltpu.VMEM((KI, CHUNK), jnp.int32),
            pltpu.VMEM((KI, CHUNK), jnp.int32),
            pltpu.VMEM((CHUNK,), jnp.int32),
            pltpu.VMEM((CHUNK,), jnp.int32),
            pltpu.VMEM((CHUNK, D), jnp.float32),
            pltpu.VMEM((CHUNK, D), jnp.float32),
            pltpu.VMEM_SHARED((N_ACC, D), jnp.float32),
            pltpu.SemaphoreType.DMA,
            pltpu.SemaphoreType.DMA,
            pltpu.SemaphoreType.DMA,
            pltpu.SemaphoreType.DMA,
        ],
    )
    def k(w_hbm, src_hbm, dst_hbm, out_hbm,
          src_v, dst_v, dstf0, dstf1, rows0, rows1, acc_sh,
          sem0, sem1, ssem0, ssem1):
        cid = lax.axis_index("c")
        sid = lax.axis_index("s")
        wid = sid * 2 + cid

        zvec = jnp.zeros((16,), jnp.float32)

        def zfill(r, carry):
            for t in range(D // 16):
                rows0[r, pl.ds(16 * t, 16)] = zvec
            return carry

        lax.fori_loop(0, CHUNK, zfill, 0)
        for q in range(4):
            pltpu.async_copy(
                rows0, acc_sh.at[pl.ds(sid * MAIN_ROWS + q * CHUNK, CHUNK)], sem0)
        pltpu.async_copy(rows0.at[pl.ds(0, MAIN_ROWS - 4 * CHUNK)],
                         acc_sh.at[pl.ds(sid * MAIN_ROWS + 4 * CHUNK,
                                         MAIN_ROWS - 4 * CHUNK)], sem1)

        @pl.when(sid < 2)
        def _():
            off = 16 * MAIN_ROWS + sid * TAIL_ROWS
            pltpu.async_copy(rows0.at[pl.ds(0, TAIL_ROWS)],
                             acc_sh.at[pl.ds(off, TAIL_ROWS)], ssem0)
            pltpu.make_async_copy(rows0.at[pl.ds(0, TAIL_ROWS)],
                                  acc_sh.at[pl.ds(off, TAIL_ROWS)], ssem0).wait()

        for q in range(4):
            pltpu.make_async_copy(
                rows0, acc_sh.at[pl.ds(sid * MAIN_ROWS + q * CHUNK, CHUNK)],
                sem0).wait()
        pltpu.make_async_copy(rows0.at[pl.ds(0, MAIN_ROWS - 4 * CHUNK)],
                              acc_sh.at[pl.ds(sid * MAIN_ROWS + 4 * CHUNK,
                                              MAIN_ROWS - 4 * CHUNK)], sem1).wait()

        base = pl.multiple_of(
            jnp.where(wid < 25, CH_A * wid, 25 * CH_A + CH_B * (wid - 25)), 8)
        n = jnp.where(wid < 25, CH_A, CH_B)

        plsc.subcore_barrier()

        for b in range(2):
            pltpu.sync_copy(src_hbm.at[pl.ds(base + b * KI, KI)], src_v)
            pltpu.sync_copy(dst_hbm.at[pl.ds(base + b * KI, KI)], dst_v)
            nb = KI if b == 0 else n - KI

            def stage_row(row, dstref):
                for t in range(CHUNK // 16):
                    dstref[pl.ds(16 * t, 16)] = dst_v[row, pl.ds(16 * t, 16)]

            pltpu.async_copy(w_hbm.at[src_v.at[0]], rows0, sem0)
            stage_row(0, dstf0)

            def body(h, carry):
                j = 2 * h
                pltpu.async_copy(w_hbm.at[src_v.at[j + 1]], rows1, sem1)
                stage_row(j + 1, dstf1)
                pltpu.make_async_copy(w_hbm.at[src_v.at[j]], rows0, sem0).wait()
                pltpu.sync_copy(rows0, acc_sh.at[dstf0], add=True)
                jn = jnp.minimum(j + 2, nb - 1)
                pltpu.async_copy(w_hbm.at[src_v.at[jn]], rows0, sem0)
                stage_row(jn, dstf0)
                pltpu.make_async_copy(w_hbm.at[src_v.at[j + 1]], rows1, sem1).wait()
                pltpu.sync_copy(rows1, acc_sh.at[dstf1], add=True)
                return carry

            lax.fori_loop(0, nb // 2, body, 0)
            pltpu.make_async_copy(w_hbm.at[src_v.at[0]], rows0, sem0).wait()

        plsc.subcore_barrier()
        pltpu.sync_copy(acc_sh.at[pl.ds(sid * MAIN_ROWS, MAIN_ROWS)],
                        out_hbm.at[cid, pl.ds(sid * MAIN_ROWS, MAIN_ROWS)])

        @pl.when(sid < 2)
        def _():
            off = 16 * MAIN_ROWS + sid * TAIL_ROWS
            pltpu.sync_copy(acc_sh.at[pl.ds(off, TAIL_ROWS)],
                            out_hbm.at[cid, pl.ds(off, TAIL_ROWS)])

    return k(weighted, src2d, dst2d)


def _combine_body(p_ref, ci_ref, o_ref):
    o_ref[...] = (p_ref[0] + p_ref[1]) * ci_ref[...]


def _combine(partials, ci):
    grid = (N_NODES // M_TILE,)
    return pl.pallas_call(
        _combine_body,
        grid=grid,
        in_specs=[
            pl.BlockSpec((2, M_TILE, D), lambda i: (0, i, 0)),
            pl.BlockSpec((M_TILE, 1), lambda i: (i, 0)),
        ],
        out_specs=pl.BlockSpec((M_TILE, D), lambda i: (i, 0)),
        out_shape=jax.ShapeDtypeStruct((N_NODES, D), jnp.float32),
    )(partials, ci)


def kernel(x, edge_index, cj, ci, W, b):
    src = jnp.concatenate(
        [edge_index[0].astype(jnp.int32),
         jnp.zeros((PAD_E,), jnp.int32)]).reshape(N_CHUNKS_PAD, CHUNK)
    dst = jnp.concatenate(
        [edge_index[1].astype(jnp.int32),
         N_NODES + (jnp.arange(PAD_E, dtype=jnp.int32) % 8)]
    ).reshape(N_CHUNKS_PAD, CHUNK)
    weighted = _feature_transform(x, W, b.reshape(1, D), cj)
    partials = _sc_scatter(weighted, src, dst)
    return _combine(partials, ci)


# speedup vs baseline: 10.7434x; 1.0135x over previous
import functools

import jax
import jax.numpy as jnp
from jax import lax
from jax.experimental import pallas as pl
from jax.experimental.pallas import tpu as pltpu
from jax.experimental.pallas import tpu_sc as plsc

N_NODES = 10000
D = 128
E = 320000

CHUNK = 128
N_CHUNKS = E // CHUNK
NW = 32
MAIN_ROWS = 624
TAIL_ROWS = 8
PAD_E = 512
N_CHUNKS_PAD = (E + PAD_E) // CHUNK
CH_A = 80
CH_B = 72
KI = 40
N_ACC = N_NODES + 8

M_TILE = 2000


def _mm_body(x_ref, w_ref, b_ref, cj_ref, o_ref):
    h = lax.dot_general(x_ref[...], w_ref[...],
                        (((1,), (1,)), ((), ())),
                        preferred_element_type=jnp.float32)
    o_ref[...] = (h + b_ref[...]) * cj_ref[...]


def _feature_transform(x, w, b, cj):
    grid = (N_NODES // M_TILE,)
    return pl.pallas_call(
        _mm_body,
        grid=grid,
        in_specs=[
            pl.BlockSpec((M_TILE, D), lambda i: (i, 0)),
            pl.BlockSpec((D, D), lambda i: (0, 0)),
            pl.BlockSpec((1, D), lambda i: (0, 0)),
            pl.BlockSpec((M_TILE, 1), lambda i: (i, 0)),
        ],
        out_specs=pl.BlockSpec((M_TILE, D), lambda i: (i, 0)),
        out_shape=jax.ShapeDtypeStruct((N_NODES, D), jnp.float32),
    )(x, w, b, cj)


def _sc_scatter(weighted, src2d, dst2d):
    mesh = plsc.VectorSubcoreMesh(core_axis_name="c", subcore_axis_name="s")

    @functools.partial(
        pl.kernel,
        mesh=mesh,
        out_type=jax.ShapeDtypeStruct((2, N_NODES, D), jnp.float32),
        scratch_types=[
            pltpu.VMEM((KI, CHUNK), jnp.int32),
            pltpu.VMEM((KI, CHUNK), jnp.int32),
            pltpu.VMEM((CHUNK,), jnp.int32),
            pltpu.VMEM((CHUNK,), jnp.int32),
            pltpu.VMEM((CHUNK, D), jnp.float32),
            pltpu.VMEM((CHUNK, D), jnp.float32),
            pltpu.VMEM_SHARED((N_ACC, D), jnp.float32),
            pltpu.SemaphoreType.DMA,
            pltpu.SemaphoreType.DMA,
            pltpu.SemaphoreType.DMA,
            pltpu.SemaphoreType.DMA,
            pltpu.SemaphoreType.DMA,
        ],
    )
    def k(w_hbm, src_hbm, dst_hbm, out_hbm,
          src_v, dst_v, dstf0, dstf1, rows0, rows1, acc_sh,
          sem0, sem1, ssem0, ssem1, sem_t):
        cid = lax.axis_index("c")
        sid = lax.axis_index("s")
        wid = sid * 2 + cid

        base = pl.multiple_of(
            jnp.where(wid < 25, CH_A * wid, 25 * CH_A + CH_B * (wid - 25)), 8)
        n = jnp.where(wid < 25, CH_A, CH_B)

        pltpu.async_copy(src_hbm.at[pl.ds(base, KI)], src_v, ssem0)
        pltpu.async_copy(dst_hbm.at[pl.ds(base, KI)], dst_v, ssem1)

        zvec = jnp.zeros((16,), jnp.float32)

        def zfill(r, carry):
            for t in range(D // 16):
                rows0[r, pl.ds(16 * t, 16)] = zvec
            return carry

        lax.fori_loop(0, CHUNK, zfill, 0)
        for q in range(4):
            pltpu.async_copy(
                rows0, acc_sh.at[pl.ds(sid * MAIN_ROWS + q * CHUNK, CHUNK)], sem0)
        pltpu.async_copy(rows0.at[pl.ds(0, MAIN_ROWS - 4 * CHUNK)],
                         acc_sh.at[pl.ds(sid * MAIN_ROWS + 4 * CHUNK,
                                         MAIN_ROWS - 4 * CHUNK)], sem1)

        @pl.when(sid < 2)
        def _():
            off = 16 * MAIN_ROWS + sid * TAIL_ROWS
            pltpu.async_copy(rows0.at[pl.ds(0, TAIL_ROWS)],
                             acc_sh.at[pl.ds(off, TAIL_ROWS)], sem_t)
            pltpu.make_async_copy(rows0.at[pl.ds(0, TAIL_ROWS)],
                                  acc_sh.at[pl.ds(off, TAIL_ROWS)], sem_t).wait()

        for q in range(4):
            pltpu.make_async_copy(
                rows0, acc_sh.at[pl.ds(sid * MAIN_ROWS + q * CHUNK, CHUNK)],
                sem0).wait()
        pltpu.make_async_copy(rows0.at[pl.ds(0, MAIN_ROWS - 4 * CHUNK)],
                              acc_sh.at[pl.ds(sid * MAIN_ROWS + 4 * CHUNK,
                                              MAIN_ROWS - 4 * CHUNK)], sem1).wait()

        plsc.subcore_barrier()

        for b in range(2):
            if b == 0:
                pltpu.make_async_copy(
                    src_hbm.at[pl.ds(base, KI)], src_v, ssem0).wait()
                pltpu.make_async_copy(
                    dst_hbm.at[pl.ds(base, KI)], dst_v, ssem1).wait()
            else:
                pltpu.async_copy(src_hbm.at[pl.ds(base + KI, KI)], src_v, ssem0)
                pltpu.async_copy(dst_hbm.at[pl.ds(base + KI, KI)], dst_v, ssem1)
                pltpu.make_async_copy(
                    src_hbm.at[pl.ds(base + KI, KI)], src_v, ssem0).wait()
                pltpu.make_async_copy(
                    dst_hbm.at[pl.ds(base + KI, KI)], dst_v, ssem1).wait()
            nb = KI if b == 0 else n - KI

            def stage_row(row, dstref):
                for t in range(CHUNK // 16):
                    dstref[pl.ds(16 * t, 16)] = dst_v[row, pl.ds(16 * t, 16)]

            pltpu.async_copy(w_hbm.at[src_v.at[0]], rows0, sem0)
            stage_row(0, dstf0)

            def body(h, carry):
                j = 2 * h
                pltpu.async_copy(w_hbm.at[src_v.at[j + 1]], rows1, sem1)
                stage_row(j + 1, dstf1)
                pltpu.make_async_copy(w_hbm.at[src_v.at[j]], rows0, sem0).wait()
                pltpu.sync_copy(rows0, acc_sh.at[dstf0], add=True)
                jn = jnp.minimum(j + 2, nb - 1)
                pltpu.async_copy(w_hbm.at[src_v.at[jn]], rows0, sem0)
                stage_row(jn, dstf0)
                pltpu.make_async_copy(w_hbm.at[src_v.at[j + 1]], rows1, sem1).wait()
                pltpu.sync_copy(rows1, acc_sh.at[dstf1], add=True)
                return carry

            lax.fori_loop(0, nb // 2, body, 0)
            pltpu.make_async_copy(w_hbm.at[src_v.at[0]], rows0, sem0).wait()

        plsc.subcore_barrier()
        pltpu.sync_copy(acc_sh.at[pl.ds(sid * MAIN_ROWS, MAIN_ROWS)],
                        out_hbm.at[cid, pl.ds(sid * MAIN_ROWS, MAIN_ROWS)])

        @pl.when(sid < 2)
        def _():
            off = 16 * MAIN_ROWS + sid * TAIL_ROWS
            pltpu.sync_copy(acc_sh.at[pl.ds(off, TAIL_ROWS)],
                            out_hbm.at[cid, pl.ds(off, TAIL_ROWS)])

    return k(weighted, src2d, dst2d)


def _combine_body(p_ref, ci_ref, o_ref):
    o_ref[...] = (p_ref[0] + p_ref[1]) * ci_ref[...]


def _combine(partials, ci):
    grid = (N_NODES // M_TILE,)
    return pl.pallas_call(
        _combine_body,
        grid=grid,
        in_specs=[
            pl.BlockSpec((2, M_TILE, D), lambda i: (0, i, 0)),
            pl.BlockSpec((M_TILE, 1), lambda i: (i, 0)),
        ],
        out_specs=pl.BlockSpec((M_TILE, D), lambda i: (i, 0)),
        out_shape=jax.ShapeDtypeStruct((N_NODES, D), jnp.float32),
    )(partials, ci)


def kernel(x, edge_index, cj, ci, W, b):
    src = jnp.concatenate(
        [edge_index[0].astype(jnp.int32),
         jnp.zeros((PAD_E,), jnp.int32)]).reshape(N_CHUNKS_PAD, CHUNK)
    dst = jnp.concatenate(
        [edge_index[1].astype(jnp.int32),
         N_NODES + (jnp.arange(PAD_E, dtype=jnp.int32) % 8)]
    ).reshape(N_CHUNKS_PAD, CHUNK)
    weighted = _feature_transform(x, W, b.reshape(1, D), cj)
    partials = _sc_scatter(weighted, src, dst)
    return _combine(partials, ci)
